# Initial kernel scaffold; baseline (speedup 1.0000x reference)
import jax
import jax.numpy as jnp
from jax.experimental import pallas as pl


def kernel(ref, query):
    raise NotImplementedError("write your pallas kernel here")


# speedup vs baseline: 14.1805x; 14.1805x over previous
import functools

import jax
import jax.numpy as jnp
from jax.experimental import pallas as pl

K_TOP = 64
LANES = 128
N_SETS = 2
R_SLOTS = 5
BQ_DIST = 512
BN_DIST = 2048
BQ_SEL = 16
BQ_MERGE = 256
NCAND = N_SETS * R_SLOTS * LANES


def _sqdist_block(q_ref, r_ref, o_ref, *, n_total, bn):
    nb = pl.program_id(1)
    q = q_ref[...]
    r = r_ref[...]
    q2 = jnp.sum(q * q, axis=1, keepdims=True)
    r2 = jnp.sum(r * r, axis=1)[None, :]
    qr = jax.lax.dot_general(q, r, (((1,), (1,)), ((), ())),
                             preferred_element_type=jnp.float32)
    sq = q2 + r2 - 2.0 * qr
    col = nb * bn + jax.lax.broadcasted_iota(jnp.int32, sq.shape, 1)
    o_ref[...] = jnp.where(col < n_total, sq, jnp.inf)


def _chain_insert(vals, idxs, x, xi):
    for i in range(R_SLOTS):
        m, mi = vals[i], idxs[i]
        swap = x < m
        vals[i] = jnp.minimum(m, x)
        idxs[i] = jnp.where(swap, xi, mi)
        x = jnp.maximum(m, x)
        xi = jnp.where(swap, mi, xi)


def _select_block(d_ref, val_ref, idx_ref, *, s_steps, bq):
    l_iota = jax.lax.broadcasted_iota(jnp.int32, (bq, LANES), 1)
    inf = jnp.full((bq, LANES), jnp.inf, jnp.float32)
    zero_i = jnp.zeros((bq, LANES), jnp.int32)

    def body(g, carry):
        va, ia, vb, ib = (list(c) for c in carry)
        t0 = g * 2
        off0 = pl.multiple_of(t0 * LANES, LANES)
        x0 = d_ref[:, pl.ds(off0, LANES)]
        off1 = pl.multiple_of((t0 + 1) * LANES, LANES)
        x1 = d_ref[:, pl.ds(off1, LANES)]
        _chain_insert(va, ia, x0, l_iota + t0 * LANES)
        _chain_insert(vb, ib, x1, l_iota + (t0 + 1) * LANES)
        return (tuple(va), tuple(ia), tuple(vb), tuple(ib))

    init = (tuple(inf for _ in range(R_SLOTS)),
            tuple(zero_i for _ in range(R_SLOTS)),
            tuple(inf for _ in range(R_SLOTS)),
            tuple(zero_i for _ in range(R_SLOTS)))
    va, ia, vb, ib = jax.lax.fori_loop(0, s_steps // 2, body, init)
    allv = list(va) + list(vb)
    alli = list(ia) + list(ib)
    for i in range(N_SETS * R_SLOTS):
        val_ref[:, i * LANES:(i + 1) * LANES] = allv[i]
        idx_ref[:, i * LANES:(i + 1) * LANES] = alli[i]


def _merge_block(val_ref, idx_ref, oval_ref, oidx_ref, *, bq):
    big = jnp.int32(2**30)

    def body(r, carry):
        v, ix = carry
        gmin = jnp.min(v, axis=1)
        eq = v == gmin[:, None]
        sel = jnp.min(jnp.where(eq, ix, big), axis=1)
        rm = eq & (ix == sel[:, None])
        v = jnp.where(rm, jnp.inf, v)
        oval_ref[pl.ds(r, 1), :] = jnp.sqrt(jnp.maximum(gmin, 0.0))[None, :]
        oidx_ref[pl.ds(r, 1), :] = sel[None, :]
        return (v, ix)

    jax.lax.fori_loop(0, K_TOP, body, (val_ref[...], idx_ref[...]))


def kernel(ref, query):
    n, dim = ref.shape
    qn = query.shape[0]
    npad = ((n + BN_DIST - 1) // BN_DIST) * BN_DIST
    s_steps = npad // LANES
    refp = jnp.pad(ref, ((0, npad - n), (0, 0)))

    sq = pl.pallas_call(
        functools.partial(_sqdist_block, n_total=n, bn=BN_DIST),
        grid=(qn // BQ_DIST, npad // BN_DIST),
        in_specs=[
            pl.BlockSpec((BQ_DIST, dim), lambda i, j: (i, 0)),
            pl.BlockSpec((BN_DIST, dim), lambda i, j: (j, 0)),
        ],
        out_specs=pl.BlockSpec((BQ_DIST, BN_DIST), lambda i, j: (i, j)),
        out_shape=jax.ShapeDtypeStruct((qn, npad), jnp.float32),
    )(query, refp)

    cval, cidx = pl.pallas_call(
        functools.partial(_select_block, s_steps=s_steps, bq=BQ_SEL),
        grid=(qn // BQ_SEL,),
        in_specs=[pl.BlockSpec((BQ_SEL, npad), lambda i: (i, 0))],
        out_specs=[
            pl.BlockSpec((BQ_SEL, NCAND), lambda i: (i, 0)),
            pl.BlockSpec((BQ_SEL, NCAND), lambda i: (i, 0)),
        ],
        out_shape=[
            jax.ShapeDtypeStruct((qn, NCAND), jnp.float32),
            jax.ShapeDtypeStruct((qn, NCAND), jnp.int32),
        ],
    )(sq)

    oval_t, oidx_t = pl.pallas_call(
        functools.partial(_merge_block, bq=BQ_MERGE),
        grid=(qn // BQ_MERGE,),
        in_specs=[
            pl.BlockSpec((BQ_MERGE, NCAND), lambda i: (i, 0)),
            pl.BlockSpec((BQ_MERGE, NCAND), lambda i: (i, 0)),
        ],
        out_specs=[
            pl.BlockSpec((K_TOP, BQ_MERGE), lambda i: (0, i)),
            pl.BlockSpec((K_TOP, BQ_MERGE), lambda i: (0, i)),
        ],
        out_shape=[
            jax.ShapeDtypeStruct((K_TOP, qn), jnp.float32),
            jax.ShapeDtypeStruct((K_TOP, qn), jnp.int32),
        ],
    )(cval, cidx)

    return (oval_t.T, oidx_t.T)


# speedup vs baseline: 15.9409x; 1.1241x over previous
import functools

import jax
import jax.numpy as jnp
from jax.experimental import pallas as pl

K_TOP = 64
LANES = 128
N_SETS = 2
R_SLOTS = 5
BQ_DIST = 512
BN_DIST = 2048
BQ_SEL = 8
BQ_MERGE = 256
NCAND = N_SETS * R_SLOTS * LANES


def _sqdist_block(q_ref, r_ref, o_ref, *, n_total, bn):
    nb = pl.program_id(1)
    q = q_ref[...]
    r = r_ref[...]
    q2 = jnp.sum(q * q, axis=1, keepdims=True)
    r2 = jnp.sum(r * r, axis=1)[None, :]
    qr = jax.lax.dot_general(q, r, (((1,), (1,)), ((), ())),
                             preferred_element_type=jnp.float32)
    sq = q2 + r2 - 2.0 * qr
    col = nb * bn + jax.lax.broadcasted_iota(jnp.int32, sq.shape, 1)
    o_ref[...] = jnp.where(col < n_total, sq, jnp.inf)


def _chain_insert(vals, idxs, x, xi):
    for i in range(R_SLOTS):
        m, mi = vals[i], idxs[i]
        swap = x < m
        vals[i] = jnp.minimum(m, x)
        idxs[i] = jnp.where(swap, xi, mi)
        x = jnp.maximum(m, x)
        xi = jnp.where(swap, mi, xi)


def _select_block(d_ref, val_ref, idx_ref, *, s_steps, bq):
    l_iota = jax.lax.broadcasted_iota(jnp.int32, (bq, LANES), 1)
    inf = jnp.full((bq, LANES), jnp.inf, jnp.float32)
    zero_i = jnp.zeros((bq, LANES), jnp.int32)

    def body(g, carry):
        va, ia, vb, ib = (list(c) for c in carry)
        t0 = g * 4
        for k in range(0, 4, 2):
            offa = pl.multiple_of((t0 + k) * LANES, LANES)
            xa = d_ref[:, pl.ds(offa, LANES)]
            offb = pl.multiple_of((t0 + k + 1) * LANES, LANES)
            xb = d_ref[:, pl.ds(offb, LANES)]
            _chain_insert(va, ia, xa, l_iota + (t0 + k) * LANES)
            _chain_insert(vb, ib, xb, l_iota + (t0 + k + 1) * LANES)
        return (tuple(va), tuple(ia), tuple(vb), tuple(ib))

    init = (tuple(inf for _ in range(R_SLOTS)),
            tuple(zero_i for _ in range(R_SLOTS)),
            tuple(inf for _ in range(R_SLOTS)),
            tuple(zero_i for _ in range(R_SLOTS)))
    va, ia, vb, ib = jax.lax.fori_loop(0, s_steps // 4, body, init)
    allv = list(va) + list(vb)
    alli = list(ia) + list(ib)
    for i in range(N_SETS * R_SLOTS):
        val_ref[:, i * LANES:(i + 1) * LANES] = allv[i]
        idx_ref[:, i * LANES:(i + 1) * LANES] = alli[i]


def _merge_block(val_ref, idx_ref, oval_ref, oidx_ref, *, bq):
    big = jnp.int32(2**30)

    def body(r, carry):
        v, ix = carry
        gmin = jnp.min(v, axis=1)
        eq = v == gmin[:, None]
        sel = jnp.min(jnp.where(eq, ix, big), axis=1)
        rm = eq & (ix == sel[:, None])
        v = jnp.where(rm, jnp.inf, v)
        oval_ref[pl.ds(r, 1), :] = jnp.sqrt(jnp.maximum(gmin, 0.0))[None, :]
        oidx_ref[pl.ds(r, 1), :] = sel[None, :]
        return (v, ix)

    jax.lax.fori_loop(0, K_TOP, body, (val_ref[...], idx_ref[...]))


def kernel(ref, query):
    n, dim = ref.shape
    qn = query.shape[0]
    npad = ((n + BN_DIST - 1) // BN_DIST) * BN_DIST
    s_steps = npad // LANES
    refp = jnp.pad(ref, ((0, npad - n), (0, 0)))

    sq = pl.pallas_call(
        functools.partial(_sqdist_block, n_total=n, bn=BN_DIST),
        grid=(qn // BQ_DIST, npad // BN_DIST),
        in_specs=[
            pl.BlockSpec((BQ_DIST, dim), lambda i, j: (i, 0)),
            pl.BlockSpec((BN_DIST, dim), lambda i, j: (j, 0)),
        ],
        out_specs=pl.BlockSpec((BQ_DIST, BN_DIST), lambda i, j: (i, j)),
        out_shape=jax.ShapeDtypeStruct((qn, npad), jnp.float32),
    )(query, refp)

    cval, cidx = pl.pallas_call(
        functools.partial(_select_block, s_steps=s_steps, bq=BQ_SEL),
        grid=(qn // BQ_SEL,),
        in_specs=[pl.BlockSpec((BQ_SEL, npad), lambda i: (i, 0))],
        out_specs=[
            pl.BlockSpec((BQ_SEL, NCAND), lambda i: (i, 0)),
            pl.BlockSpec((BQ_SEL, NCAND), lambda i: (i, 0)),
        ],
        out_shape=[
            jax.ShapeDtypeStruct((qn, NCAND), jnp.float32),
            jax.ShapeDtypeStruct((qn, NCAND), jnp.int32),
        ],
    )(sq)

    oval_t, oidx_t = pl.pallas_call(
        functools.partial(_merge_block, bq=BQ_MERGE),
        grid=(qn // BQ_MERGE,),
        in_specs=[
            pl.BlockSpec((BQ_MERGE, NCAND), lambda i: (i, 0)),
            pl.BlockSpec((BQ_MERGE, NCAND), lambda i: (i, 0)),
        ],
        out_specs=[
            pl.BlockSpec((K_TOP, BQ_MERGE), lambda i: (0, i)),
            pl.BlockSpec((K_TOP, BQ_MERGE), lambda i: (0, i)),
        ],
        out_shape=[
            jax.ShapeDtypeStruct((K_TOP, qn), jnp.float32),
            jax.ShapeDtypeStruct((K_TOP, qn), jnp.int32),
        ],
    )(cval, cidx)

    return (oval_t.T, oidx_t.T)
